# baseline (speedup 1.0000x reference)
import jax
import jax.numpy as jnp
import numpy as np
from jax import lax
from jax.experimental import pallas as pl
from jax.experimental.pallas import tpu as pltpu

B = 1024
N_FIELDS = 26
EMB_DIM = 32
D_CAT = 224
TRIU = D_CAT * (D_CAT + 1) // 2
CHUNK = 512
N_CHUNKS = 49
STREAM_END = N_CHUNKS * CHUNK


def _row_off(i):
    return D_CAT * i - (i * (i - 1)) // 2


def _segments(k0, k1):
    segs = []
    for i in range(D_CAT):
        lo = max(_row_off(i), k0)
        hi = min(_row_off(i) + D_CAT - i, k1)
        if lo < hi:
            segs.append((i, i + (lo - _row_off(i)), i + (hi - _row_off(i))))
    return segs


def _select_mats():
    su = np.zeros((STREAM_END, D_CAT), np.float32)
    sv = np.zeros((STREAM_END, D_CAT), np.float32)
    k = 0
    for i in range(D_CAT):
        w = D_CAT - i
        for j in range(i, D_CAT):
            if k >= STREAM_END:
                break
            su[k, i] = 1.0
            sv[k, j] = 1.0
            k += 1
    return su, sv


_SU, _SV = _select_mats()


def _dot_t(x, w):
    return lax.dot_general(x, w, (((1,), (1,)), ((), ())),
                           preferred_element_type=jnp.float32)


def _prologue_body(rows_ref, xe_ref, xd_ref,
                   pj_w1_ref, pj_b1_ref, pj_w2_ref, pj_b2_ref,
                   pj_g_ref, pj_bt_ref,
                   bm_w1_ref, bm_b1_ref, bm_w2_ref, bm_b2_ref,
                   w1tail_ref, tm_b1_ref,
                   xemb_ref, c_ref, acc0_ref):
    h = jnp.maximum(_dot_t(xe_ref[...], pj_w1_ref[...]) + pj_b1_ref[...], 0.0)
    h = _dot_t(h, pj_w2_ref[...]) + pj_b2_ref[...]
    mean = jnp.mean(h, axis=0, keepdims=True)
    var = jnp.mean((h - mean) * (h - mean), axis=0, keepdims=True)
    x_embed = (pj_g_ref[...] * (h - mean) * lax.rsqrt(var + 1e-5)
               + pj_bt_ref[...])
    xemb_ref[...] = x_embed

    bot = jnp.maximum(_dot_t(xd_ref[...], bm_w1_ref[...]) + bm_b1_ref[...],
                      0.0)
    bot = jnp.maximum(_dot_t(bot, bm_w2_ref[...]) + bm_b2_ref[...], 0.0)

    r_mod = jax.lax.broadcasted_iota(jnp.int32,
                                     (N_FIELDS * EMB_DIM, EMB_DIM), 0)
    c_id = jax.lax.broadcasted_iota(jnp.int32,
                                    (N_FIELDS * EMB_DIM, EMB_DIM), 1)
    sel = (r_mod % EMB_DIM == c_id).astype(jnp.float32)
    embed_x = lax.dot_general(rows_ref[...], sel, (((1,), (0,)), ((), ())),
                              preferred_element_type=jnp.float32)

    c = jnp.concatenate([bot, x_embed, embed_x], axis=1)
    c_ref[...] = c

    segs = [c[:, i:i + 1] * c[:, jl:jh]
            for i, jl, jh in _segments(STREAM_END, TRIU)]
    segs.append(bot)
    tail_prod = jnp.concatenate(segs, axis=1)
    acc0_ref[...] = _dot_t(tail_prod, w1tail_ref[...]) + tm_b1_ref[...]


def _dot_hilo(x, w):
    xh = x.astype(jnp.bfloat16)
    xl = (x - xh.astype(jnp.float32)).astype(jnp.bfloat16)
    wh = w.astype(jnp.bfloat16)
    wl = (w - wh.astype(jnp.float32)).astype(jnp.bfloat16)
    return _dot_t(xh, wh) + _dot_t(xh, wl) + _dot_t(xl, wh)


def _interact_body(c_ref, su_ref, sv_ref, w1_ref, acc_ref):
    g = pl.program_id(0)
    c = c_ref[...]
    ch = c.astype(jnp.bfloat16)
    cl = (c - ch.astype(jnp.float32)).astype(jnp.bfloat16)
    su = su_ref[...]
    sv = sv_ref[...]
    cu = _dot_t(ch, su) + _dot_t(cl, su)
    cv = _dot_t(ch, sv) + _dot_t(cl, sv)
    d = _dot_hilo(cu * cv, w1_ref[...])

    @pl.when(g == 0)
    def _init():
        acc_ref[...] = d

    @pl.when(g != 0)
    def _accum():
        acc_ref[...] = acc_ref[...] + d


def _epilogue_body(acc_ref, acc0_ref, tm_w2_ref, tm_b2_ref,
                   tm_w3_ref, tm_b3_ref, out_ref):
    t = jnp.maximum(acc_ref[...] + acc0_ref[...], 0.0)
    t = jnp.maximum(_dot_t(t, tm_w2_ref[...]) + tm_b2_ref[...], 0.0)
    logit = _dot_t(t, tm_w3_ref[...])[:, 0:1] + tm_b3_ref[0, 0]
    out_ref[...] = jax.nn.sigmoid(logit)


def kernel(x_sparse, x_dense, x_embed_before_projection, emb_table,
           pj_w1, pj_b1, pj_w2, pj_b2, pj_gamma, pj_beta,
           bm_w1, bm_b1, bm_w2, bm_b2,
           tm_w1, tm_b1, tm_w2, tm_b2, tm_w3, tm_b3):
    rows = jnp.take(emb_table, x_sparse.astype(jnp.int32).reshape(-1),
                    axis=0)
    rows832 = rows.reshape(B, N_FIELDS * EMB_DIM)

    vmem = pl.BlockSpec(memory_space=pltpu.VMEM)

    xemb, c, acc0 = pl.pallas_call(
        _prologue_body,
        out_shape=(jax.ShapeDtypeStruct((B, 64), jnp.float32),
                   jax.ShapeDtypeStruct((B, D_CAT), jnp.float32),
                   jax.ShapeDtypeStruct((B, 512), jnp.float32)),
        in_specs=[vmem] * 15,
        out_specs=(vmem, vmem, vmem),
    )(rows832, x_embed_before_projection, x_dense,
      pj_w1, pj_b1.reshape(1, -1), pj_w2, pj_b2.reshape(1, -1),
      pj_gamma.reshape(1, -1), pj_beta.reshape(1, -1),
      bm_w1, bm_b1.reshape(1, -1), bm_w2, bm_b2.reshape(1, -1),
      tm_w1[:, STREAM_END:], tm_b1.reshape(1, -1))

    def full(shape):
        nd = len(shape)
        return pl.BlockSpec(shape, lambda g, _nd=nd: (0,) * _nd)

    acc = pl.pallas_call(
        _interact_body,
        grid=(N_CHUNKS,),
        out_shape=jax.ShapeDtypeStruct((B, 512), jnp.float32),
        in_specs=[
            full((B, D_CAT)),
            pl.BlockSpec((CHUNK, D_CAT), lambda g: (g, 0)),
            pl.BlockSpec((CHUNK, D_CAT), lambda g: (g, 0)),
            pl.BlockSpec((512, CHUNK), lambda g: (0, g)),
        ],
        out_specs=full((B, 512)),
    )(c, jnp.asarray(_SU, jnp.bfloat16), jnp.asarray(_SV, jnp.bfloat16),
      tm_w1)

    out, = pl.pallas_call(
        _epilogue_body,
        out_shape=(jax.ShapeDtypeStruct((B, 1), jnp.float32),),
        in_specs=[vmem, vmem, vmem, vmem, vmem,
                  pl.BlockSpec(memory_space=pltpu.SMEM)],
        out_specs=(vmem,),
    )(acc, acc0, tm_w2, tm_b2.reshape(1, -1),
      jnp.pad(tm_w3, ((0, 7), (0, 0))), tm_b3.reshape(1, 1))
    return (out, xemb)


# speedup vs baseline: 1.3944x; 1.3944x over previous
import jax
import jax.numpy as jnp
import numpy as np
from jax import lax
from jax.experimental import pallas as pl
from jax.experimental.pallas import tpu as pltpu

B = 1024
N_FIELDS = 26
EMB_DIM = 32
D_CAT = 224
TRIU = D_CAT * (D_CAT + 1) // 2
CHUNK = 512
N_CHUNKS = 49
STREAM_END = N_CHUNKS * CHUNK


def _row_off(i):
    return D_CAT * i - (i * (i - 1)) // 2


def _segments(k0, k1):
    segs = []
    for i in range(D_CAT):
        lo = max(_row_off(i), k0)
        hi = min(_row_off(i) + D_CAT - i, k1)
        if lo < hi:
            segs.append((i, i + (lo - _row_off(i)), i + (hi - _row_off(i))))
    return segs


def _select_mats():
    su = np.zeros((STREAM_END, D_CAT), np.float32)
    sv = np.zeros((STREAM_END, D_CAT), np.float32)
    k = 0
    for i in range(D_CAT):
        w = D_CAT - i
        for j in range(i, D_CAT):
            if k >= STREAM_END:
                break
            su[k, i] = 1.0
            sv[k, j] = 1.0
            k += 1
    return su, sv


_SU, _SV = _select_mats()


def _dot_t(x, w):
    return lax.dot_general(x, w, (((1,), (1,)), ((), ())),
                           preferred_element_type=jnp.float32)


def _prologue_body(rows_ref, xe_ref, xd_ref,
                   pj_w1_ref, pj_b1_ref, pj_w2_ref, pj_b2_ref,
                   pj_g_ref, pj_bt_ref,
                   bm_w1_ref, bm_b1_ref, bm_w2_ref, bm_b2_ref,
                   w1tail_ref, tm_b1_ref,
                   xemb_ref, c_ref, acc0_ref):
    h = jnp.maximum(_dot_t(xe_ref[...], pj_w1_ref[...]) + pj_b1_ref[...], 0.0)
    h = _dot_t(h, pj_w2_ref[...]) + pj_b2_ref[...]
    mean = jnp.mean(h, axis=0, keepdims=True)
    var = jnp.mean((h - mean) * (h - mean), axis=0, keepdims=True)
    x_embed = (pj_g_ref[...] * (h - mean) * lax.rsqrt(var + 1e-5)
               + pj_bt_ref[...])
    xemb_ref[...] = x_embed

    bot = jnp.maximum(_dot_t(xd_ref[...], bm_w1_ref[...]) + bm_b1_ref[...],
                      0.0)
    bot = jnp.maximum(_dot_t(bot, bm_w2_ref[...]) + bm_b2_ref[...], 0.0)

    r_mod = jax.lax.broadcasted_iota(jnp.int32,
                                     (N_FIELDS * EMB_DIM, EMB_DIM), 0)
    c_id = jax.lax.broadcasted_iota(jnp.int32,
                                    (N_FIELDS * EMB_DIM, EMB_DIM), 1)
    sel = (r_mod % EMB_DIM == c_id).astype(jnp.float32)
    embed_x = lax.dot_general(rows_ref[...], sel, (((1,), (0,)), ((), ())),
                              preferred_element_type=jnp.float32)

    c = jnp.concatenate([bot, x_embed, embed_x], axis=1)
    c_ref[...] = c

    segs = [c[:, i:i + 1] * c[:, jl:jh]
            for i, jl, jh in _segments(STREAM_END, TRIU)]
    segs.append(bot)
    tail_prod = jnp.concatenate(segs, axis=1)
    acc0_ref[...] = _dot_t(tail_prod, w1tail_ref[...]) + tm_b1_ref[...]


def _interact_body(c_ref, su_ref, sv_ref, w1_ref, acc_ref):
    g = pl.program_id(0)
    c = c_ref[...]
    cu = _dot_t(c, su_ref[...])
    cv = _dot_t(c, sv_ref[...])
    d = _dot_t(cu * cv, w1_ref[...])

    @pl.when(g == 0)
    def _init():
        acc_ref[...] = d

    @pl.when(g != 0)
    def _accum():
        acc_ref[...] = acc_ref[...] + d


def _epilogue_body(acc_ref, acc0_ref, tm_w2_ref, tm_b2_ref,
                   tm_w3_ref, tm_b3_ref, out_ref):
    t = jnp.maximum(acc_ref[...] + acc0_ref[...], 0.0)
    t = jnp.maximum(_dot_t(t, tm_w2_ref[...]) + tm_b2_ref[...], 0.0)
    logit = _dot_t(t, tm_w3_ref[...])[:, 0:1] + tm_b3_ref[0, 0]
    out_ref[...] = jax.nn.sigmoid(logit)


def kernel(x_sparse, x_dense, x_embed_before_projection, emb_table,
           pj_w1, pj_b1, pj_w2, pj_b2, pj_gamma, pj_beta,
           bm_w1, bm_b1, bm_w2, bm_b2,
           tm_w1, tm_b1, tm_w2, tm_b2, tm_w3, tm_b3):
    rows = jnp.take(emb_table, x_sparse.astype(jnp.int32).reshape(-1),
                    axis=0)
    rows832 = rows.reshape(B, N_FIELDS * EMB_DIM)

    vmem = pl.BlockSpec(memory_space=pltpu.VMEM)

    xemb, c, acc0 = pl.pallas_call(
        _prologue_body,
        out_shape=(jax.ShapeDtypeStruct((B, 64), jnp.float32),
                   jax.ShapeDtypeStruct((B, D_CAT), jnp.float32),
                   jax.ShapeDtypeStruct((B, 512), jnp.float32)),
        in_specs=[vmem] * 15,
        out_specs=(vmem, vmem, vmem),
    )(rows832, x_embed_before_projection, x_dense,
      pj_w1, pj_b1.reshape(1, -1), pj_w2, pj_b2.reshape(1, -1),
      pj_gamma.reshape(1, -1), pj_beta.reshape(1, -1),
      bm_w1, bm_b1.reshape(1, -1), bm_w2, bm_b2.reshape(1, -1),
      tm_w1[:, STREAM_END:], tm_b1.reshape(1, -1))

    def full(shape):
        nd = len(shape)
        return pl.BlockSpec(shape, lambda g, _nd=nd: (0,) * _nd)

    acc = pl.pallas_call(
        _interact_body,
        grid=(N_CHUNKS,),
        out_shape=jax.ShapeDtypeStruct((B, 512), jnp.float32),
        in_specs=[
            full((B, D_CAT)),
            pl.BlockSpec((CHUNK, D_CAT), lambda g: (g, 0)),
            pl.BlockSpec((CHUNK, D_CAT), lambda g: (g, 0)),
            pl.BlockSpec((512, CHUNK), lambda g: (0, g)),
        ],
        out_specs=full((B, 512)),
    )(c, jnp.asarray(_SU), jnp.asarray(_SV), tm_w1)

    out, = pl.pallas_call(
        _epilogue_body,
        out_shape=(jax.ShapeDtypeStruct((B, 1), jnp.float32),),
        in_specs=[vmem, vmem, vmem, vmem, vmem,
                  pl.BlockSpec(memory_space=pltpu.SMEM)],
        out_specs=(vmem,),
    )(acc, acc0, tm_w2, tm_b2.reshape(1, -1),
      jnp.pad(tm_w3, ((0, 7), (0, 0))), tm_b3.reshape(1, 1))
    return (out, xemb)
